# Initial kernel scaffold; baseline (speedup 1.0000x reference)
import jax
import jax.numpy as jnp
from jax.experimental import pallas as pl


def kernel(feature, gamma, fc_w, fc_b):
    raise NotImplementedError("write your pallas kernel here")


# speedup vs baseline: 2.4805x; 2.4805x over previous
import functools

import numpy as np
import jax
import jax.numpy as jnp
from jax.experimental import pallas as pl
from jax.experimental.pallas import tpu as pltpu

_HIGHEST = jax.lax.Precision.HIGHEST


def _dot(a, b):
    return jax.lax.dot_general(a, b, (((1,), (0,)), ((), ())),
                               precision=_HIGHEST,
                               preferred_element_type=jnp.float32)


def _dot_t(a, b):
    return jax.lax.dot_general(a, b, (((1,), (1,)), ((), ())),
                               precision=_HIGHEST,
                               preferred_element_type=jnp.float32)


def _comp_body(f_ref, out_ref, acc_sum, acc_max, *, nc):
    c = pl.program_id(1)
    x = f_ref[0]
    s = jnp.sum(x, axis=0)
    m = jnp.max(x, axis=0)

    @pl.when(c == 0)
    def _():
        acc_sum[...] = s
        acc_max[...] = m

    @pl.when(c > 0)
    def _():
        acc_sum[...] += s
        acc_max[...] = jnp.maximum(acc_max[...], m)

    @pl.when(c == pl.num_programs(1) - 1)
    def _():
        out_ref[0] = acc_sum[...] * (1.0 / nc) + acc_max[...]


def _dft_body(x_ref, fr_ref, fi_ref, yr_ref, yi_ref, amp_ref):
    x = x_ref[0]
    fr = fr_ref[...]
    fi = fi_ref[...]
    tr = _dot(fr, x)
    ti = _dot(fi, x)
    yr = _dot_t(tr, fr) - _dot_t(ti, fi)
    yi = _dot_t(tr, fi) + _dot_t(ti, fr)
    yr_ref[0] = yr
    yi_ref[0] = yi
    amp_ref[0] = jnp.sqrt(yr * yr + yi * yi)


def _seg_body(amp_ref, oh_ref, w_ref, b_ref, fre_ref, acc):
    p = pl.program_id(0)
    part = _dot(amp_ref[...], oh_ref[...])

    @pl.when(p == 0)
    def _():
        acc[...] = part

    @pl.when(p > 0)
    def _():
        acc[...] += part

    @pl.when(p == pl.num_programs(0) - 1)
    def _():
        z = _dot(acc[...], w_ref[...]) + b_ref[...]
        fre_ref[...] = jnp.where(z >= 0.0, z, 0.01 * z)


def _gather_body(fre_ref, oh_ref, att_ref):
    att_ref[...] = _dot_t(fre_ref[...], oh_ref[...])


def _idft_body(yr_ref, yi_ref, att_ref, gr_ref, gi_ref, out_ref):
    att = att_ref[0]
    mr = yr_ref[0] * att
    mi = yi_ref[0] * att
    gr = gr_ref[...]
    gi = gi_ref[...]
    ur = _dot(gr, mr) - _dot(gi, mi)
    ui = _dot(gr, mi) + _dot(gi, mr)
    nfm = _dot_t(ur, gr) - _dot_t(ui, gi)
    mn = jnp.min(nfm)
    mx = jnp.max(nfm)
    out_ref[0] = (nfm - mn) * (1.0 / (mx - mn))


def _scale_body(f_ref, a_ref, g_ref, out_ref):
    g = g_ref[...]
    a = a_ref[0]
    out_ref[0] = f_ref[0] * (1.0 + g * a[None, :, :])


@functools.lru_cache(maxsize=2)
def _static_tables(H, W):
    N = H
    F = np.fft.fft(np.eye(N))
    Fs = np.fft.fftshift(F, axes=0)
    G = np.fft.fftshift(np.conj(F) / N, axes=1)
    center_h, center_w = H // 2, W // 2
    R = min(center_h, center_w)
    hh = np.arange(H) - center_h
    ww = np.arange(W) - center_w
    r = np.sqrt(hh[:, None] ** 2 + ww[None, :] ** 2)
    labels = np.minimum(np.floor(r), R).astype(np.int64)
    nlab = R + 1
    counts = np.bincount(labels.reshape(-1), minlength=nlab).astype(np.float64)
    KPAD = 128
    onehot = np.zeros((H * W, KPAD), np.float32)
    onehot[np.arange(H * W), labels.reshape(-1)] = 1.0
    return (Fs.real.astype(np.float32), Fs.imag.astype(np.float32),
            G.real.astype(np.float32), G.imag.astype(np.float32),
            labels, counts, nlab, KPAD, onehot)


def kernel(feature, gamma, fc_w, fc_b):
    B, C, H, W = feature.shape
    fsr, fsi, gr, gi, labels, counts, nlab, KPAD, onehot = _static_tables(H, W)
    P = H * W

    inv_c = jnp.asarray(1.0 / counts, jnp.float32)
    w2j = jnp.zeros((KPAD, KPAD), jnp.float32)
    w2j = w2j.at[:nlab, :nlab].set(inv_c[:, None] * fc_w.T)
    b2j = jnp.zeros((1, KPAD), jnp.float32)
    b2j = b2j.at[0, :nlab].set(fc_b)

    CBLK = 32
    NC = C // CBLK

    comp = pl.pallas_call(
        functools.partial(_comp_body, nc=C),
        grid=(B, NC),
        in_specs=[pl.BlockSpec((1, CBLK, H, W), lambda b, c: (b, c, 0, 0))],
        out_specs=pl.BlockSpec((1, H, W), lambda b, c: (b, 0, 0)),
        out_shape=jax.ShapeDtypeStruct((B, H, W), jnp.float32),
        scratch_shapes=[pltpu.VMEM((H, W), jnp.float32),
                        pltpu.VMEM((H, W), jnp.float32)],
    )(feature)

    fsr_j = jnp.asarray(fsr)
    fsi_j = jnp.asarray(fsi)
    gr_j = jnp.asarray(gr)
    gi_j = jnp.asarray(gi)

    full = lambda b: (0, 0)
    yr, yi, amp = pl.pallas_call(
        _dft_body,
        grid=(B,),
        in_specs=[pl.BlockSpec((1, H, W), lambda b: (b, 0, 0)),
                  pl.BlockSpec((H, W), full),
                  pl.BlockSpec((H, W), full)],
        out_specs=[pl.BlockSpec((1, H, W), lambda b: (b, 0, 0))] * 3,
        out_shape=[jax.ShapeDtypeStruct((B, H, W), jnp.float32)] * 3,
    )(comp, fsr_j, fsi_j)

    amp_f = amp.reshape(B, P)
    oh_j = jnp.asarray(onehot)
    NP = 8
    PBLK = P // NP
    fre = pl.pallas_call(
        _seg_body,
        grid=(NP,),
        in_specs=[pl.BlockSpec((B, PBLK), lambda p: (0, p)),
                  pl.BlockSpec((PBLK, KPAD), lambda p: (p, 0)),
                  pl.BlockSpec((KPAD, KPAD), lambda p: (0, 0)),
                  pl.BlockSpec((1, KPAD), lambda p: (0, 0))],
        out_specs=pl.BlockSpec((B, KPAD), lambda p: (0, 0)),
        out_shape=jax.ShapeDtypeStruct((B, KPAD), jnp.float32),
        scratch_shapes=[pltpu.VMEM((B, KPAD), jnp.float32)],
    )(amp_f, oh_j, w2j, b2j)
    att_f = pl.pallas_call(
        _gather_body,
        grid=(NP,),
        in_specs=[pl.BlockSpec((B, KPAD), lambda p: (0, 0)),
                  pl.BlockSpec((PBLK, KPAD), lambda p: (p, 0))],
        out_specs=pl.BlockSpec((B, PBLK), lambda p: (0, p)),
        out_shape=jax.ShapeDtypeStruct((B, P), jnp.float32),
    )(fre, oh_j)
    att = att_f.reshape(B, H, W)

    attn = pl.pallas_call(
        _idft_body,
        grid=(B,),
        in_specs=[pl.BlockSpec((1, H, W), lambda b: (b, 0, 0)),
                  pl.BlockSpec((1, H, W), lambda b: (b, 0, 0)),
                  pl.BlockSpec((1, H, W), lambda b: (b, 0, 0)),
                  pl.BlockSpec((H, W), full),
                  pl.BlockSpec((H, W), full)],
        out_specs=pl.BlockSpec((1, H, W), lambda b: (b, 0, 0)),
        out_shape=jax.ShapeDtypeStruct((B, H, W), jnp.float32),
    )(yr, yi, att, gr_j, gi_j)

    gamma3 = gamma.reshape(C, 1, 1)
    out = pl.pallas_call(
        _scale_body,
        grid=(B, NC),
        in_specs=[pl.BlockSpec((1, CBLK, H, W), lambda b, c: (b, c, 0, 0)),
                  pl.BlockSpec((1, H, W), lambda b, c: (b, 0, 0)),
                  pl.BlockSpec((CBLK, 1, 1), lambda b, c: (c, 0, 0))],
        out_specs=pl.BlockSpec((1, CBLK, H, W), lambda b, c: (b, c, 0, 0)),
        out_shape=jax.ShapeDtypeStruct((B, C, H, W), jnp.float32),
    )(feature, attn, gamma3)
    return out
